# baseline (speedup 1.0000x reference)
import functools

import jax
import jax.numpy as jnp
from jax import lax
from jax.experimental import pallas as pl
from jax.experimental.pallas import tpu as pltpu
from jax.experimental.pallas import tpu_sc as plsc

N = 10000
E = 320000
D = 128

NC = 2
NS = 16
NW = NC * NS
L = 16

EPAD = 327680
EP = EPAD // NW
K = 128
SCH = 20
NSUP = EP // (K * SCH)

NPAD = 10240
NP = NPAD // NS
PC = 32
NPC = NP // PC

TB = 1024


def _sc_kernels():
    mesh = plsc.VectorSubcoreMesh(
        core_axis_name="c", subcore_axis_name="s",
        num_cores=NC, num_subcores=NS)

    def _zero_acc(node0, zstage, acc_sp):
        zeros16 = jnp.zeros((L,), jnp.float32)

        def _zb(i, _):
            for v in range(D // L):
                zstage[i, pl.ds(v * L, L)] = zeros16
            return 0
        lax.fori_loop(0, PC, _zb, 0)

        def _zs(t, _):
            sl = pl.ds(pl.multiple_of(node0 + t * PC, PC), PC)
            pltpu.sync_copy(zstage, acc_sp.at[sl])
            return 0
        lax.fori_loop(0, NPC, _zs, 0)

    def _acc_out(c, node0, zstage, acc_sp, out_h):
        def _wb(t, _):
            sl = pl.ds(pl.multiple_of(node0 + t * PC, PC), PC)
            pltpu.sync_copy(acc_sp.at[sl], zstage)
            pltpu.sync_copy(zstage, out_h.at[c, sl])
            return 0
        lax.fori_loop(0, NPC, _wb, 0)

    @functools.partial(
        pl.kernel, mesh=mesh,
        out_type=jax.ShapeDtypeStruct((NC, NPAD, D), jnp.float32),
        scratch_types=[
            pltpu.VMEM_SHARED((NPAD, D), jnp.float32),
            pltpu.VMEM((SCH, K), jnp.int32),
            pltpu.VMEM((K, D), jnp.float32),
            pltpu.VMEM((PC, D), jnp.float32),
            pltpu.SemaphoreType.DMA,
        ])
    def histogram(cols4, out_h, acc_sp, cols_sc, ones_b, zstage, sems):
        c = lax.axis_index("c")
        s = lax.axis_index("s")
        wid = s * NC + c
        node0 = s * NP

        ones16 = jnp.ones((L,), jnp.float32)

        def _fo(i, _):
            for v in range(D // L):
                ones_b[i, pl.ds(v * L, L)] = ones16
            return 0
        lax.fori_loop(0, K, _fo, 0)
        _zero_acc(node0, zstage, acc_sp)
        plsc.subcore_barrier()

        def _sup(sup, _):
            pltpu.sync_copy(cols4.at[wid, sup], cols_sc)
            descs = [
                pltpu.async_copy(ones_b, acc_sp.at[cols_sc.at[j]], sems,
                                 add=True)
                for j in range(SCH)
            ]
            for d in descs:
                d.wait()
            return 0
        lax.fori_loop(0, NSUP, _sup, 0)
        plsc.subcore_barrier()
        _acc_out(c, node0, zstage, acc_sp, out_h)

    @functools.partial(
        pl.kernel, mesh=mesh,
        out_type=jax.ShapeDtypeStruct((NC, NPAD, D), jnp.float32),
        scratch_types=[
            pltpu.VMEM_SHARED((NPAD, D), jnp.float32),
            pltpu.VMEM((SCH, K), jnp.int32),
            pltpu.VMEM((SCH, K), jnp.int32),
            pltpu.VMEM((K, D), jnp.float32),
            pltpu.VMEM((K, D), jnp.float32),
            pltpu.VMEM((PC, D), jnp.float32),
            pltpu.SemaphoreType.DMA,
            pltpu.SemaphoreType.DMA,
        ])
    def propagate(xs_hbm, rows4, cols4, out_h,
                  acc_sp, rows_sc, cols_sc, stage0, stage1, zstage,
                  semg, sems):
        c = lax.axis_index("c")
        s = lax.axis_index("s")
        wid = s * NC + c
        node0 = s * NP

        _zero_acc(node0, zstage, acc_sp)
        plsc.subcore_barrier()

        def _sup(sup, _):
            pltpu.sync_copy(rows4.at[wid, sup], rows_sc)
            pltpu.sync_copy(cols4.at[wid, sup], cols_sc)
            def _edges(j, _):
                pltpu.async_copy(xs_hbm.at[rows_sc.at[j]], stage0,
                                 semg).wait()
                pltpu.sync_copy(stage0, acc_sp.at[cols_sc.at[j]], add=True)
                return 0
            lax.fori_loop(0, SCH, _edges, 0)
            return 0
        lax.fori_loop(0, NSUP, _sup, 0)
        plsc.subcore_barrier()
        _acc_out(c, node0, zstage, acc_sp, out_h)

    return histogram, propagate


_histogram, _propagate = _sc_kernels()


def _prescale_tc(p0, p1, x, o_xs, o_dis):
    dis = lax.rsqrt(p0[0] + p1[0] + 1.0)
    o_dis[...] = dis
    o_xs[...] = dis * x[...]


def _combine_tc(q0, q1, xs, dis, bias, o):
    o[...] = dis[...] * (q0[0] + q1[0] + xs[...]) + bias[...]


@jax.jit
def kernel(x, edge_index, bias):
    xpad = jnp.pad(x, ((0, NPAD - N), (0, 0)))
    npad_e = EPAD - E
    pad_rows = jnp.zeros((npad_e,), jnp.int32)
    pad_cols = N + (jnp.arange(npad_e, dtype=jnp.int32) % (NPAD - N))
    rows4 = jnp.concatenate([edge_index[0], pad_rows]).reshape(
        NW, NSUP, SCH, K)
    cols4 = jnp.concatenate([edge_index[1], pad_cols]).reshape(
        NW, NSUP, SCH, K)

    deg_parts = _histogram(cols4)

    blk = pl.BlockSpec((TB, D), lambda i: (i, 0))
    xs, dis = pl.pallas_call(
        _prescale_tc,
        grid=(NPAD // TB,),
        in_specs=[pl.BlockSpec((1, TB, D), lambda i: (0, i, 0)),
                  pl.BlockSpec((1, TB, D), lambda i: (1, i, 0)),
                  blk],
        out_specs=[blk, blk],
        out_shape=[jax.ShapeDtypeStruct((NPAD, D), jnp.float32),
                   jax.ShapeDtypeStruct((NPAD, D), jnp.float32)],
    )(deg_parts, deg_parts, xpad)

    acc_parts = _propagate(xs, rows4, cols4)

    out = pl.pallas_call(
        _combine_tc,
        grid=(NPAD // TB,),
        in_specs=[pl.BlockSpec((1, TB, D), lambda i: (0, i, 0)),
                  pl.BlockSpec((1, TB, D), lambda i: (1, i, 0)),
                  blk, blk,
                  pl.BlockSpec((1, D), lambda i: (0, 0))],
        out_specs=blk,
        out_shape=jax.ShapeDtypeStruct((NPAD, D), jnp.float32),
    )(acc_parts, acc_parts, xs, dis, bias.reshape(1, D))

    return (out[:N], x)


# speedup vs baseline: 1.0750x; 1.0750x over previous
import functools

import jax
import jax.numpy as jnp
from jax import lax
from jax.experimental import pallas as pl
from jax.experimental.pallas import tpu as pltpu
from jax.experimental.pallas import tpu_sc as plsc

N = 10000
E = 320000
D = 128

NC = 2
NS = 16
NW = NC * NS
L = 16

EPAD = 327680
EP = EPAD // NW
K = 128
SCH = 20
NSUP = EP // (K * SCH)

NPAD = 10240
NP = NPAD // NS
PC = 32
NPC = NP // PC

TB = 1024


def _sc_kernels():
    mesh = plsc.VectorSubcoreMesh(
        core_axis_name="c", subcore_axis_name="s",
        num_cores=NC, num_subcores=NS)

    def _zero_acc(node0, zstage, acc_sp):
        zeros16 = jnp.zeros((L,), jnp.float32)

        def _zb(i, _):
            for v in range(D // L):
                zstage[i, pl.ds(v * L, L)] = zeros16
            return 0
        lax.fori_loop(0, PC, _zb, 0)

        def _zs(t, _):
            sl = pl.ds(pl.multiple_of(node0 + t * PC, PC), PC)
            pltpu.sync_copy(zstage, acc_sp.at[sl])
            return 0
        lax.fori_loop(0, NPC, _zs, 0)

    def _acc_out(c, node0, zstage, acc_sp, out_h):
        def _wb(t, _):
            sl = pl.ds(pl.multiple_of(node0 + t * PC, PC), PC)
            pltpu.sync_copy(acc_sp.at[sl], zstage)
            pltpu.sync_copy(zstage, out_h.at[c, sl])
            return 0
        lax.fori_loop(0, NPC, _wb, 0)

    @functools.partial(
        pl.kernel, mesh=mesh,
        out_type=jax.ShapeDtypeStruct((NC, NPAD, D), jnp.float32),
        scratch_types=[
            pltpu.VMEM_SHARED((NPAD, D), jnp.float32),
            pltpu.VMEM((SCH, K), jnp.int32),
            pltpu.VMEM((K, D), jnp.float32),
            pltpu.VMEM((PC, D), jnp.float32),
            pltpu.SemaphoreType.DMA,
        ])
    def histogram(cols4, out_h, acc_sp, cols_sc, ones_b, zstage, sems):
        c = lax.axis_index("c")
        s = lax.axis_index("s")
        wid = s * NC + c
        node0 = s * NP

        ones16 = jnp.ones((L,), jnp.float32)

        def _fo(i, _):
            for v in range(D // L):
                ones_b[i, pl.ds(v * L, L)] = ones16
            return 0
        lax.fori_loop(0, K, _fo, 0)
        _zero_acc(node0, zstage, acc_sp)
        plsc.subcore_barrier()

        def _sup(sup, _):
            pltpu.sync_copy(cols4.at[wid, sup], cols_sc)
            descs = [
                pltpu.async_copy(ones_b, acc_sp.at[cols_sc.at[j]], sems,
                                 add=True)
                for j in range(SCH)
            ]
            for d in descs:
                d.wait()
            return 0
        lax.fori_loop(0, NSUP, _sup, 0)
        plsc.subcore_barrier()
        _acc_out(c, node0, zstage, acc_sp, out_h)

    @functools.partial(
        pl.kernel, mesh=mesh,
        out_type=jax.ShapeDtypeStruct((NC, NPAD, D), jnp.float32),
        scratch_types=[
            pltpu.VMEM_SHARED((NPAD, D), jnp.float32),
            pltpu.VMEM((SCH, K), jnp.int32),
            pltpu.VMEM((SCH, K), jnp.int32),
            pltpu.VMEM((K, D), jnp.float32),
            pltpu.VMEM((K, D), jnp.float32),
            pltpu.VMEM((PC, D), jnp.float32),
            pltpu.SemaphoreType.DMA,
            pltpu.SemaphoreType.DMA,
            pltpu.SemaphoreType.DMA,
        ])
    def propagate(xs_hbm, rows4, cols4, out_h,
                  acc_sp, rows_sc, cols_sc, stage0, stage1, zstage,
                  semg, sems0, sems1):
        c = lax.axis_index("c")
        s = lax.axis_index("s")
        wid = s * NC + c
        node0 = s * NP

        _zero_acc(node0, zstage, acc_sp)
        plsc.subcore_barrier()

        stages = (stage0, stage1)
        ssems = (sems0, sems1)

        def _sup(sup, _):
            pltpu.sync_copy(rows4.at[wid, sup], rows_sc)
            pltpu.sync_copy(cols4.at[wid, sup], cols_sc)
            pend = [None, None]
            for j in range(SCH):
                b = j & 1
                if pend[b] is not None:
                    pend[b].wait()
                pltpu.async_copy(xs_hbm.at[rows_sc.at[j]], stages[b],
                                 semg).wait()
                pend[b] = pltpu.async_copy(stages[b], acc_sp.at[cols_sc.at[j]],
                                           ssems[b], add=True)
            pend[0].wait()
            pend[1].wait()
            return 0
        lax.fori_loop(0, NSUP, _sup, 0)
        plsc.subcore_barrier()
        _acc_out(c, node0, zstage, acc_sp, out_h)

    return histogram, propagate


_histogram, _propagate = _sc_kernels()


def _prescale_tc(p0, p1, x, o_xs, o_dis):
    dis = lax.rsqrt(p0[0] + p1[0] + 1.0)
    o_dis[...] = dis
    o_xs[...] = dis * x[...]


def _combine_tc(q0, q1, xs, dis, bias, o):
    o[...] = dis[...] * (q0[0] + q1[0] + xs[...]) + bias[...]


@jax.jit
def kernel(x, edge_index, bias):
    xpad = jnp.pad(x, ((0, NPAD - N), (0, 0)))
    npad_e = EPAD - E
    pad_rows = jnp.zeros((npad_e,), jnp.int32)
    pad_cols = N + (jnp.arange(npad_e, dtype=jnp.int32) % (NPAD - N))
    rows4 = jnp.concatenate([edge_index[0], pad_rows]).reshape(
        NW, NSUP, SCH, K)
    cols4 = jnp.concatenate([edge_index[1], pad_cols]).reshape(
        NW, NSUP, SCH, K)

    deg_parts = _histogram(cols4)

    blk = pl.BlockSpec((TB, D), lambda i: (i, 0))
    xs, dis = pl.pallas_call(
        _prescale_tc,
        grid=(NPAD // TB,),
        in_specs=[pl.BlockSpec((1, TB, D), lambda i: (0, i, 0)),
                  pl.BlockSpec((1, TB, D), lambda i: (1, i, 0)),
                  blk],
        out_specs=[blk, blk],
        out_shape=[jax.ShapeDtypeStruct((NPAD, D), jnp.float32),
                   jax.ShapeDtypeStruct((NPAD, D), jnp.float32)],
    )(deg_parts, deg_parts, xpad)

    acc_parts = _propagate(xs, rows4, cols4)

    out = pl.pallas_call(
        _combine_tc,
        grid=(NPAD // TB,),
        in_specs=[pl.BlockSpec((1, TB, D), lambda i: (0, i, 0)),
                  pl.BlockSpec((1, TB, D), lambda i: (1, i, 0)),
                  blk, blk,
                  pl.BlockSpec((1, D), lambda i: (0, 0))],
        out_specs=blk,
        out_shape=jax.ShapeDtypeStruct((NPAD, D), jnp.float32),
    )(acc_parts, acc_parts, xs, dis, bias.reshape(1, D))

    return (out[:N], x)
